# baseline (speedup 1.0000x reference)
import functools

import jax
import jax.numpy as jnp
from jax import lax
from jax.experimental import pallas as pl
from jax.experimental.pallas import tpu as pltpu
from jax.experimental.pallas import tpu_sc as plsc

VOCAB_ROWS = 1000000
HID = 64
L = 200
EPS = 1e-12
NBUF = 4
C1, C2 = 104, 96
BLK = 128
NW = 32

FULL_BLOCKS = VOCAB_ROWS // BLK
TAIL = VOCAB_ROWS - FULL_BLOCKS * BLK


def _sc_transpose(wt, tail):
    mesh = plsc.VectorSubcoreMesh(core_axis_name="c", subcore_axis_name="s")
    iters = FULL_BLOCKS // NW + 1

    @functools.partial(
        pl.kernel,
        out_type=jax.ShapeDtypeStruct((VOCAB_ROWS * HID // BLK, BLK), jnp.float32),
        mesh=mesh,
        scratch_types=[
            [pltpu.VMEM((HID, BLK), jnp.float32) for _ in range(2)],
            [pltpu.VMEM((HID, BLK), jnp.float32) for _ in range(2)],
            [pltpu.SemaphoreType.DMA for _ in range(2)],
            [pltpu.SemaphoreType.DMA for _ in range(2)],
        ],
        compiler_params=pltpu.CompilerParams(
            needs_layout_passes=False, use_tc_tiling_on_sc=True
        ),
    )
    def k(wt_hbm, tail_hbm, out_hbm, slabs, outs, isems, osems):
        wid = lax.axis_index("s") * 2 + lax.axis_index("c")
        lane = lax.iota(jnp.int32, 16)
        rows_k = [k16 * 16 + lane for k16 in range(HID // 16)]

        def fire_in(bid, b):
            pltpu.async_copy(
                wt_hbm.at[:, pl.ds(bid * BLK, BLK)], slabs[b], isems[b]
            )

        def wait_in(b):
            pltpu.make_async_copy(
                wt_hbm.at[:, pl.ds(0, BLK)], slabs[b], isems[b]
            ).wait()

        def fire_out(bid, b):
            pltpu.async_copy(
                outs[b], out_hbm.at[pl.ds(bid * HID, HID), :], osems[b]
            )

        def wait_out(b):
            pltpu.make_async_copy(
                out_hbm.at[pl.ds(0, HID), :], outs[b], osems[b]
            ).wait()

        def transpose_block(b):
            @plsc.parallel_loop(0, BLK // 2, unroll=16)
            def _cols(m):
                for half in range(2):
                    col = jnp.zeros((16,), jnp.int32) + (m * 2 + half)
                    for k16 in range(HID // 16):
                        vals = plsc.load_gather(slabs[b], [rows_k[k16], col])
                        outs[b][m, pl.ds(half * HID + k16 * 16, 16)] = vals

        fire_in(wid, 0)

        @pl.loop(0, (iters + 1) // 2)
        def _main(g):
            for p in range(2):
                i = g * 2 + p
                bid = wid + i * NW
                nbid = bid + NW

                @pl.when(nbid < FULL_BLOCKS)
                def _():
                    fire_in(nbid, 1 - p)

                @pl.when(bid < FULL_BLOCKS)
                def _():
                    wait_in(p)

                    @pl.when(i >= 2)
                    def _():
                        wait_out(p)

                    transpose_block(p)
                    fire_out(bid, p)

        wait_out(0)
        wait_out(1)

        @pl.when(wid == NW - 1)
        def _():
            pltpu.sync_copy(tail_hbm, slabs[0])

            @plsc.parallel_loop(0, TAIL // 2, unroll=8)
            def _cols(m):
                for half in range(2):
                    col = jnp.zeros((16,), jnp.int32) + (m * 2 + half)
                    for k16 in range(HID // 16):
                        vals = plsc.load_gather(slabs[0], [rows_k[k16], col])
                        outs[0][m, pl.ds(half * HID + k16 * 16, 16)] = vals

            pltpu.sync_copy(
                outs[0].at[pl.ds(0, TAIL // 2), :],
                out_hbm.at[pl.ds(FULL_BLOCKS * HID, TAIL // 2), :],
            )

    return k(wt, tail)


def _sc_pool(xs, xt, w_lin, n_side):
    rows_per_w = n_side // 16
    mesh = plsc.VectorSubcoreMesh(core_axis_name="c", subcore_axis_name="s")

    @functools.partial(
        pl.kernel,
        out_type=(
            jax.ShapeDtypeStruct((n_side * HID,), jnp.float32),
            jax.ShapeDtypeStruct((n_side * HID,), jnp.float32),
            jax.ShapeDtypeStruct((n_side * 16,), jnp.float32),
            jax.ShapeDtypeStruct((n_side * 16,), jnp.float32),
        ),
        mesh=mesh,
        scratch_types=[
            pltpu.VMEM((rows_per_w, L), jnp.int32),
            [pltpu.VMEM((L, HID), jnp.float32) for _ in range(NBUF)],
            pltpu.VMEM((rows_per_w * HID,), jnp.float32),
            pltpu.VMEM((rows_per_w * 16,), jnp.float32),
            [pltpu.SemaphoreType.DMA for _ in range(NBUF)],
        ],
        compiler_params=pltpu.CompilerParams(use_tc_tiling_on_sc=False),
    )
    def k(xs_hbm, xt_hbm, w_hbm, os_hbm, ot_hbm, cs_hbm, ct_hbm,
          idx_v, bufs, pooled_v, cnts_v, sems):
        wid = lax.axis_index("s") * 2 + lax.axis_index("c")

        def fire(r, buf, sem):
            pltpu.async_copy(
                w_hbm.at[idx_v.at[r, pl.ds(0, C1)]], buf.at[pl.ds(0, C1)], sem
            )
            pltpu.async_copy(
                w_hbm.at[idx_v.at[r, pl.ds(C1, C2)]],
                buf.at[pl.ds(C1, C2)],
                sem,
            )

        def wait_buf(buf, sem):
            pltpu.make_async_copy(w_hbm.at[pl.ds(0, L)], buf, sem).wait()

        lane = lax.iota(jnp.int32, 16)

        def process(r, buf):
            cnt = jnp.zeros((16,), jnp.float32)
            for j in range(L // 16):
                v = idx_v[r, pl.ds(j * 16, 16)]
                cnt = cnt + jnp.where(v != 0, 1.0, 0.0).astype(jnp.float32)
            v = idx_v[r, pl.ds(L - 16, 16)]
            tail_ok = (v != 0) & (lane >= 16 - L % 16)
            cnt = cnt + jnp.where(tail_ok, 1.0, 0.0).astype(jnp.float32)
            cnts_v[pl.ds(r * 16, 16)] = cnt

            zero = jnp.zeros((16,), jnp.float32)

            @pl.loop(0, L, init_carry=(zero, zero, zero, zero), unroll=4)
            def _sum(j, carry):
                a0, a1, a2, a3 = carry
                a0 = a0 + buf[j, pl.ds(0, 16)]
                a1 = a1 + buf[j, pl.ds(16, 16)]
                a2 = a2 + buf[j, pl.ds(32, 16)]
                a3 = a3 + buf[j, pl.ds(48, 16)]
                return (a0, a1, a2, a3)

            a0, a1, a2, a3 = _sum
            ob = r * HID
            pooled_v[pl.ds(ob, 16)] = a0
            pooled_v[pl.ds(ob + 16, 16)] = a1
            pooled_v[pl.ds(ob + 32, 16)] = a2
            pooled_v[pl.ds(ob + 48, 16)] = a3

        def side(x_hbm, out_hbm, cnt_hbm, sw):
            pltpu.sync_copy(x_hbm.at[pl.ds(sw * rows_per_w, rows_per_w), :], idx_v)
            for b in range(NBUF):
                fire(b, bufs[b], sems[b])

            @pl.loop(0, rows_per_w // NBUF)
            def _outer(g):
                for b in range(NBUF):
                    r = g * NBUF + b
                    wait_buf(bufs[b], sems[b])
                    process(r, bufs[b])

                    @pl.when(r + NBUF < rows_per_w)
                    def _():
                        fire(r + NBUF, bufs[b], sems[b])

            pltpu.sync_copy(
                pooled_v,
                out_hbm.at[pl.ds(sw * rows_per_w * HID, rows_per_w * HID)],
            )
            pltpu.sync_copy(
                cnts_v, cnt_hbm.at[pl.ds(sw * rows_per_w * 16, rows_per_w * 16)]
            )

        @pl.when(wid < 16)
        def _():
            side(xs_hbm, os_hbm, cs_hbm, wid)

        @pl.when(wid >= 16)
        def _():
            side(xt_hbm, ot_hbm, ct_hbm, wid - 16)

    return k(xs, xt, w_lin)


def _tc_layernorm(pooled_s, pooled_t, cnts_s, cnts_t, gamma, beta):
    def body(ps_ref, pt_ref, cs_ref, ct_ref, g_ref, b_ref, os_ref, ot_ref):
        g = g_ref[...]
        b = b_ref[...]
        for p_ref, c_ref, o_ref in (
            (ps_ref, cs_ref, os_ref),
            (pt_ref, ct_ref, ot_ref),
        ):
            cnt = jnp.sum(c_ref[...], axis=1, keepdims=True)
            x = p_ref[...] / cnt
            mu = jnp.mean(x, axis=1, keepdims=True)
            d = x - mu
            var = jnp.mean(d * d, axis=1, keepdims=True)
            o_ref[...] = d * lax.rsqrt(var + EPS) * g + b

    n = pooled_s.shape[0]
    return pl.pallas_call(
        body,
        out_shape=(
            jax.ShapeDtypeStruct((n, HID), jnp.float32),
            jax.ShapeDtypeStruct((n, HID), jnp.float32),
        ),
    )(pooled_s, pooled_t, cnts_s, cnts_t,
      gamma.reshape(1, HID), beta.reshape(1, HID))


def kernel(x_s, x_t, W, gamma, beta):
    B = x_s.shape[0]
    wt = W.T
    tail = jnp.pad(
        lax.slice(wt, (0, FULL_BLOCKS * BLK), (HID, VOCAB_ROWS)),
        ((0, 0), (0, BLK - TAIL)),
    )
    w_lin = _sc_transpose(wt, tail).reshape(VOCAB_ROWS, HID)
    ps, pt, cs, ct = _sc_pool(
        x_s.astype(jnp.int32), x_t.astype(jnp.int32), w_lin, B
    )
    out_s, out_t = _tc_layernorm(
        ps.reshape(B, HID),
        pt.reshape(B, HID),
        cs.reshape(B, 16),
        ct.reshape(B, 16),
        gamma,
        beta,
    )
    return out_s, out_t


# speedup vs baseline: 2.2606x; 2.2606x over previous
import functools

import jax
import jax.numpy as jnp
from jax import lax
from jax.experimental import pallas as pl
from jax.experimental.pallas import tpu as pltpu
from jax.experimental.pallas import tpu_sc as plsc

VOCAB_ROWS = 1000000
HID = 64
L = 200
EPS = 1e-12
NBUF = 4
C1, C2 = 104, 96
BLK = 128
NW = 32

FULL_BLOCKS = VOCAB_ROWS // BLK
TAIL = VOCAB_ROWS - FULL_BLOCKS * BLK


def _sc_transpose(wt, tail):
    mesh = plsc.VectorSubcoreMesh(core_axis_name="c", subcore_axis_name="s")
    iters = FULL_BLOCKS // NW + 1

    @functools.partial(
        pl.kernel,
        out_type=jax.ShapeDtypeStruct((VOCAB_ROWS * HID // BLK, BLK), jnp.float32),
        mesh=mesh,
        scratch_types=[
            [pltpu.VMEM((HID, BLK), jnp.float32) for _ in range(2)],
            [pltpu.VMEM((HID, BLK), jnp.float32) for _ in range(2)],
            [pltpu.SemaphoreType.DMA for _ in range(2)],
            [pltpu.SemaphoreType.DMA for _ in range(2)],
        ],
        compiler_params=pltpu.CompilerParams(
            needs_layout_passes=False, use_tc_tiling_on_sc=True
        ),
    )
    def k(wt_hbm, tail_hbm, out_hbm, slabs, outs, isems, osems):
        wid = lax.axis_index("s") * 2 + lax.axis_index("c")
        lane = lax.iota(jnp.int32, 16)
        rows_k = [k16 * 16 + lane for k16 in range(HID // 16)]

        def fire_in(bid, b):
            pltpu.async_copy(
                wt_hbm.at[:, pl.ds(bid * BLK, BLK)], slabs[b], isems[b]
            )

        def wait_in(b):
            pltpu.make_async_copy(
                wt_hbm.at[:, pl.ds(0, BLK)], slabs[b], isems[b]
            ).wait()

        def fire_out(bid, b):
            pltpu.async_copy(
                outs[b], out_hbm.at[pl.ds(bid * HID, HID), :], osems[b]
            )

        def wait_out(b):
            pltpu.make_async_copy(
                out_hbm.at[pl.ds(0, HID), :], outs[b], osems[b]
            ).wait()

        def transpose_tile(b, n_cols):
            @plsc.parallel_loop(0, 16, unroll=2)
            def _diag(d):
                rot = (lane + d) & 15
                rot_half = lax.shift_right_logical(rot, 1)
                lpar = (rot & 1) * 64 + lane
                lidx = [lpar + r0 for r0 in range(0, HID, 16)]
                for c0 in range(0, n_cols, 16):
                    cols = rot + c0
                    midx = rot_half + (c0 // 2)
                    for k16 in range(HID // 16):
                        vals = plsc.load_gather(slabs[b], [rows_k[k16], cols])
                        plsc.store_scatter(outs[b], [midx, lidx[k16]], vals)

        def transpose_block(b):
            transpose_tile(b, BLK)

        fire_in(wid, 0)

        @pl.loop(0, (iters + 1) // 2)
        def _main(g):
            for p in range(2):
                i = g * 2 + p
                bid = wid + i * NW
                nbid = bid + NW

                @pl.when(nbid < FULL_BLOCKS)
                def _():
                    fire_in(nbid, 1 - p)

                @pl.when(bid < FULL_BLOCKS)
                def _():
                    wait_in(p)

                    @pl.when(i >= 2)
                    def _():
                        wait_out(p)

                    transpose_block(p)
                    fire_out(bid, p)

        wait_out(0)
        wait_out(1)

        @pl.when(wid == NW - 1)
        def _():
            pltpu.sync_copy(tail_hbm, slabs[0])

            transpose_tile(0, TAIL)

            pltpu.sync_copy(
                outs[0].at[pl.ds(0, TAIL // 2), :],
                out_hbm.at[pl.ds(FULL_BLOCKS * HID, TAIL // 2), :],
            )

    return k(wt, tail)


def _sc_pool(xs, xt, w_lin, n_side):
    rows_per_w = n_side // 16
    mesh = plsc.VectorSubcoreMesh(core_axis_name="c", subcore_axis_name="s")

    @functools.partial(
        pl.kernel,
        out_type=(
            jax.ShapeDtypeStruct((n_side * HID,), jnp.float32),
            jax.ShapeDtypeStruct((n_side * HID,), jnp.float32),
            jax.ShapeDtypeStruct((n_side * 16,), jnp.float32),
            jax.ShapeDtypeStruct((n_side * 16,), jnp.float32),
        ),
        mesh=mesh,
        scratch_types=[
            pltpu.VMEM((rows_per_w, L), jnp.int32),
            [pltpu.VMEM((L, HID), jnp.float32) for _ in range(NBUF)],
            pltpu.VMEM((rows_per_w * HID,), jnp.float32),
            pltpu.VMEM((rows_per_w * 16,), jnp.float32),
            [pltpu.SemaphoreType.DMA for _ in range(NBUF)],
        ],
        compiler_params=pltpu.CompilerParams(use_tc_tiling_on_sc=False),
    )
    def k(xs_hbm, xt_hbm, w_hbm, os_hbm, ot_hbm, cs_hbm, ct_hbm,
          idx_v, bufs, pooled_v, cnts_v, sems):
        wid = lax.axis_index("s") * 2 + lax.axis_index("c")

        def fire(r, buf, sem):
            pltpu.async_copy(
                w_hbm.at[idx_v.at[r, pl.ds(0, C1)]], buf.at[pl.ds(0, C1)], sem
            )
            pltpu.async_copy(
                w_hbm.at[idx_v.at[r, pl.ds(C1, C2)]],
                buf.at[pl.ds(C1, C2)],
                sem,
            )

        def wait_buf(buf, sem):
            pltpu.make_async_copy(w_hbm.at[pl.ds(0, L)], buf, sem).wait()

        lane = lax.iota(jnp.int32, 16)

        def process(r, buf):
            cnt = jnp.zeros((16,), jnp.float32)
            for j in range(L // 16):
                v = idx_v[r, pl.ds(j * 16, 16)]
                cnt = cnt + jnp.where(v != 0, 1.0, 0.0).astype(jnp.float32)
            v = idx_v[r, pl.ds(L - 16, 16)]
            tail_ok = (v != 0) & (lane >= 16 - L % 16)
            cnt = cnt + jnp.where(tail_ok, 1.0, 0.0).astype(jnp.float32)
            cnts_v[pl.ds(r * 16, 16)] = cnt

            zero = jnp.zeros((16,), jnp.float32)

            @pl.loop(0, L, init_carry=(zero, zero, zero, zero), unroll=4)
            def _sum(j, carry):
                a0, a1, a2, a3 = carry
                a0 = a0 + buf[j, pl.ds(0, 16)]
                a1 = a1 + buf[j, pl.ds(16, 16)]
                a2 = a2 + buf[j, pl.ds(32, 16)]
                a3 = a3 + buf[j, pl.ds(48, 16)]
                return (a0, a1, a2, a3)

            a0, a1, a2, a3 = _sum
            ob = r * HID
            pooled_v[pl.ds(ob, 16)] = a0
            pooled_v[pl.ds(ob + 16, 16)] = a1
            pooled_v[pl.ds(ob + 32, 16)] = a2
            pooled_v[pl.ds(ob + 48, 16)] = a3

        def side(x_hbm, out_hbm, cnt_hbm, sw):
            pltpu.sync_copy(x_hbm.at[pl.ds(sw * rows_per_w, rows_per_w), :], idx_v)
            for b in range(NBUF):
                fire(b, bufs[b], sems[b])

            @pl.loop(0, rows_per_w // NBUF)
            def _outer(g):
                for b in range(NBUF):
                    r = g * NBUF + b
                    wait_buf(bufs[b], sems[b])
                    process(r, bufs[b])

                    @pl.when(r + NBUF < rows_per_w)
                    def _():
                        fire(r + NBUF, bufs[b], sems[b])

            pltpu.sync_copy(
                pooled_v,
                out_hbm.at[pl.ds(sw * rows_per_w * HID, rows_per_w * HID)],
            )
            pltpu.sync_copy(
                cnts_v, cnt_hbm.at[pl.ds(sw * rows_per_w * 16, rows_per_w * 16)]
            )

        @pl.when(wid < 16)
        def _():
            side(xs_hbm, os_hbm, cs_hbm, wid)

        @pl.when(wid >= 16)
        def _():
            side(xt_hbm, ot_hbm, ct_hbm, wid - 16)

    return k(xs, xt, w_lin)


def _tc_layernorm(pooled_s, pooled_t, cnts_s, cnts_t, gamma, beta):
    def body(ps_ref, pt_ref, cs_ref, ct_ref, g_ref, b_ref, os_ref, ot_ref):
        g = g_ref[...]
        b = b_ref[...]
        for p_ref, c_ref, o_ref in (
            (ps_ref, cs_ref, os_ref),
            (pt_ref, ct_ref, ot_ref),
        ):
            cnt = jnp.sum(c_ref[...], axis=1, keepdims=True)
            x = p_ref[...] / cnt
            mu = jnp.mean(x, axis=1, keepdims=True)
            d = x - mu
            var = jnp.mean(d * d, axis=1, keepdims=True)
            o_ref[...] = d * lax.rsqrt(var + EPS) * g + b

    n = pooled_s.shape[0]
    return pl.pallas_call(
        body,
        out_shape=(
            jax.ShapeDtypeStruct((n, HID), jnp.float32),
            jax.ShapeDtypeStruct((n, HID), jnp.float32),
        ),
    )(pooled_s, pooled_t, cnts_s, cnts_t,
      gamma.reshape(1, HID), beta.reshape(1, HID))


def kernel(x_s, x_t, W, gamma, beta):
    B = x_s.shape[0]
    wt = W.T
    tail = jnp.pad(
        lax.slice(wt, (0, FULL_BLOCKS * BLK), (HID, VOCAB_ROWS)),
        ((0, 0), (0, BLK - TAIL)),
    )
    w_lin = _sc_transpose(wt, tail).reshape(VOCAB_ROWS, HID)
    ps, pt, cs, ct = _sc_pool(
        x_s.astype(jnp.int32), x_t.astype(jnp.int32), w_lin, B
    )
    out_s, out_t = _tc_layernorm(
        ps.reshape(B, HID),
        pt.reshape(B, HID),
        cs.reshape(B, 16),
        ct.reshape(B, 16),
        gamma,
        beta,
    )
    return out_s, out_t
